# baseline (speedup 1.0000x reference)
import functools

import jax
import jax.numpy as jnp
from jax.experimental import pallas as pl
from jax.experimental.pallas import tpu as pltpu
from jax.experimental.pallas import tpu_sc as plsc

_NC = 2
_NS = 16
_NW = _NC * _NS
_CHUNK = 8
_NBUF = 3


def _gather_rows(x, valid_idx, n_rows, d):
    per_w = n_rows // _NW
    n_chunks = per_w // _CHUNK
    mesh = plsc.VectorSubcoreMesh(core_axis_name="core",
                                  subcore_axis_name="subcore")

    @functools.partial(
        pl.kernel,
        out_type=jax.ShapeDtypeStruct((n_rows, d), x.dtype),
        mesh=mesh,
        scratch_types=[
            pltpu.VMEM((per_w,), jnp.int32),
            pltpu.VMEM((_NBUF, _CHUNK, d), x.dtype),
            pltpu.SemaphoreType.DMA,
            pltpu.SemaphoreType.DMA,
        ],
    )
    def gather_kernel(x_hbm, i_hbm, o_hbm, idx_v, buf, sem_in, sem_out):
        wid = jax.lax.axis_index("subcore") * _NC + jax.lax.axis_index("core")
        base = wid * per_w
        pltpu.sync_copy(i_hbm.at[pl.ds(base, per_w)], idx_v)

        def gather(c):
            return pltpu.make_async_copy(
                x_hbm.at[idx_v.at[pl.ds(c * _CHUNK, _CHUNK)]],
                buf.at[c % _NBUF], sem_in)

        def writeback(c):
            return pltpu.make_async_copy(
                buf.at[c % _NBUF], o_hbm.at[pl.ds(base + c * _CHUNK, _CHUNK)],
                sem_out)

        for c in range(min(_NBUF - 1, n_chunks)):
            gather(c).start()
        pending_wb = 0
        for c in range(n_chunks):
            gather(c).wait()
            writeback(c).start()
            pending_wb += 1
            nxt = c + _NBUF - 1
            if nxt < n_chunks:
                if pending_wb > _NBUF - 2:
                    writeback(c - 1).wait()
                    pending_wb -= 1
                gather(nxt).start()
        for _ in range(pending_wb):
            writeback(n_chunks - 1).wait()

    return gather_kernel(x, valid_idx)


_TC_ROWS_PER_BLOCK = 16


def _tc_gather_rows(x, idx_tc, n_tc, d):
    r = _TC_ROWS_PER_BLOCK

    def body(idx_ref, x_any, o_ref, sem):
        i = pl.program_id(0)
        for j in range(r):
            pltpu.make_async_copy(
                x_any.at[pl.ds(idx_ref[i * r + j], 1)],
                o_ref.at[pl.ds(j, 1)], sem).start()
        for j in range(r):
            pltpu.make_async_copy(
                x_any.at[pl.ds(0, 1)], o_ref.at[pl.ds(j, 1)], sem).wait()

    return pl.pallas_call(
        body,
        grid_spec=pltpu.PrefetchScalarGridSpec(
            num_scalar_prefetch=1,
            grid=(n_tc // r,),
            in_specs=[pl.BlockSpec(memory_space=pl.ANY)],
            out_specs=pl.BlockSpec((r, d), lambda i, idx_ref: (i, 0)),
            scratch_shapes=[pltpu.SemaphoreType.DMA],
        ),
        out_shape=jax.ShapeDtypeStruct((n_tc, d), x.dtype),
    )(idx_tc, x)


_N_TC = 1024


def kernel(x, valid_idx):
    n_rows = valid_idx.shape[0]
    d = x.shape[1]
    n_sc = n_rows - _N_TC
    sc_part = _gather_rows(x, valid_idx[:n_sc], n_sc, d)
    tc_part = _tc_gather_rows(x, valid_idx[n_sc:], _N_TC, d)
    return jnp.concatenate([sc_part, tc_part], axis=0)


# speedup vs baseline: 2.3152x; 2.3152x over previous
import functools

import jax
import jax.numpy as jnp
from jax.experimental import pallas as pl
from jax.experimental.pallas import tpu as pltpu
from jax.experimental.pallas import tpu_sc as plsc

_NC = 2
_NS = 16
_NW = _NC * _NS
_CHUNK = 8
_NBUF = 3


def _gather_rows(x, valid_idx, n_rows, d):
    per_w = n_rows // _NW
    n_chunks = per_w // _CHUNK
    mesh = plsc.VectorSubcoreMesh(core_axis_name="core",
                                  subcore_axis_name="subcore")

    @functools.partial(
        pl.kernel,
        out_type=jax.ShapeDtypeStruct((n_rows, d), x.dtype),
        mesh=mesh,
        scratch_types=[
            pltpu.VMEM((per_w,), jnp.int32),
            pltpu.VMEM((_NBUF, _CHUNK, d), x.dtype),
            pltpu.SemaphoreType.DMA,
            pltpu.SemaphoreType.DMA,
        ],
    )
    def gather_kernel(x_hbm, i_hbm, o_hbm, idx_v, buf, sem_in, sem_out):
        wid = jax.lax.axis_index("subcore") * _NC + jax.lax.axis_index("core")
        base = wid * per_w
        pltpu.sync_copy(i_hbm.at[pl.ds(base, per_w)], idx_v)

        def gather(c):
            return pltpu.make_async_copy(
                x_hbm.at[idx_v.at[pl.ds(c * _CHUNK, _CHUNK)]],
                buf.at[c % _NBUF], sem_in)

        def writeback(c):
            return pltpu.make_async_copy(
                buf.at[c % _NBUF], o_hbm.at[pl.ds(base + c * _CHUNK, _CHUNK)],
                sem_out)

        for c in range(min(_NBUF - 1, n_chunks)):
            gather(c).start()
        pending_wb = 0
        for c in range(n_chunks):
            gather(c).wait()
            writeback(c).start()
            pending_wb += 1
            nxt = c + _NBUF - 1
            if nxt < n_chunks:
                if pending_wb > _NBUF - 2:
                    writeback(c - 1).wait()
                    pending_wb -= 1
                gather(nxt).start()
        for _ in range(pending_wb):
            writeback(n_chunks - 1).wait()

    return gather_kernel(x, valid_idx)


def kernel(x, valid_idx):
    n_rows = valid_idx.shape[0]
    d = x.shape[1]
    return _gather_rows(x, valid_idx, n_rows, d)
